# baseline (speedup 1.0000x reference)
import functools

import jax
import jax.numpy as jnp
from jax import lax
from jax.experimental import pallas as pl
from jax.experimental.pallas import tpu as pltpu
from jax.experimental.pallas import tpu_sc as plsc

N = 10000
E = 320000
D = 128
EPS = 1e-5

NB = 10
BR = N // NB
NW = 32
ER = 5120
RPW = ER // NW
NPAD = 10016
RPS = NPAD // 16

_mesh = plsc.VectorSubcoreMesh(core_axis_name="c", subcore_axis_name="s")


DH = D // 2


def _sc_scatter(with_deg):
    out_type = [jax.ShapeDtypeStruct((2, NW, RPS, DH), jnp.float32)]
    scratch = [
        pltpu.VMEM_SHARED((NPAD, DH), jnp.float32),
        pltpu.VMEM((RPW, D), jnp.int32),
        pltpu.VMEM((RPW, D), jnp.int32),
        pltpu.VMEM((D, DH), jnp.float32),
        pltpu.VMEM((D, DH), jnp.float32),
        pltpu.VMEM((D, DH), jnp.float32),
        pltpu.VMEM((D, DH), jnp.float32),
        pltpu.SemaphoreType.DMA,
        pltpu.SemaphoreType.DMA,
        pltpu.SemaphoreType.DMA,
        pltpu.SemaphoreType.DMA,
        pltpu.SemaphoreType.DMA,
        pltpu.SemaphoreType.DMA,
        pltpu.SemaphoreType.DMA,
        pltpu.SemaphoreType.DMA,
    ]
    if with_deg:
        out_type.append(jax.ShapeDtypeStruct((NW, RPS, 16), jnp.float32))
        scratch += [
            pltpu.VMEM_SHARED((NPAD, 16), jnp.float32),
            pltpu.VMEM((D, 16), jnp.float32),
            pltpu.SemaphoreType.DMA,
            pltpu.SemaphoreType.DMA,
            pltpu.SemaphoreType.DMA,
            pltpu.SemaphoreType.DMA,
        ]

    def body(vw1a, vw1b, srcs, dsts, zeros, zeros16, ones_in, part, *rest):
        if with_deg:
            (degpart, acc_sh, src_b, dst_b, r0, r1, r2, r3,
             g0, g1, g2, g3, s0, s1, s2, s3, deg_sh, ones_v, d0, d1, d2, d3) = rest
            dsem = [d0, d1, d2, d3]
        else:
            (acc_sh, src_b, dst_b, r0, r1, r2, r3,
             g0, g1, g2, g3, s0, s1, s2, s3) = rest
        rows = [r0, r1, r2, r3]
        gsem = [g0, g1, g2, g3]
        ssem = [s0, s1, s2, s3]
        cid = lax.axis_index("c")
        sid = lax.axis_index("s")
        w = cid * 16 + sid
        base = w * RPW
        pltpu.sync_copy(srcs.at[pl.ds(base, RPW)], src_b)
        pltpu.sync_copy(dsts.at[pl.ds(base, RPW)], dst_b)
        if with_deg:
            pltpu.sync_copy(zeros16, deg_sh.at[pl.ds(sid * RPS, RPS)])
            pltpu.sync_copy(ones_in, ones_v)

        for half, vw1 in ((0, vw1a), (1, vw1b)):
            deg_pass = with_deg and half == 0

            pltpu.sync_copy(zeros, acc_sh.at[pl.ds(sid * RPS, RPS)])
            plsc.subcore_barrier()

            def gather_start(t, b):
                pltpu.async_copy(vw1.at[src_b.at[t]], rows[b], gsem[b])

            def gather_wait(t, b):
                pltpu.make_async_copy(vw1.at[src_b.at[t]], rows[b], gsem[b]).wait()

            def scatter_start(t, b):
                pltpu.async_copy(rows[b], acc_sh.at[dst_b.at[t]], ssem[b], add=True)
                if deg_pass:
                    pltpu.async_copy(ones_v, deg_sh.at[dst_b.at[t]], dsem[b], add=True)

            def scatter_wait(t, b):
                pltpu.make_async_copy(rows[b], acc_sh.at[dst_b.at[t]], ssem[b]).wait()
                if deg_pass:
                    pltpu.make_async_copy(ones_v, deg_sh.at[dst_b.at[t]], dsem[b]).wait()

            for b in range(4):
                gather_start(b, b)
            for b in range(2):
                gather_wait(b, b)
                scatter_start(b, b)

            def chunk(c, carry):
                for b in range(4):
                    t = 4 * c + b
                    b2 = (b + 2) % 4
                    gather_wait(t - 2, b2)
                    scatter_start(t - 2, b2)
                    scatter_wait(t - 4, b)
                    gather_start(t, b)
                return carry

            lax.fori_loop(1, RPW // 4, chunk, 0)

            last = RPW - 4
            for b in range(2, 4):
                gather_wait(last + b, b)
                scatter_start(last + b, b)
            for b in range(4):
                scatter_wait(last + b, b)

            plsc.subcore_barrier()
            pltpu.sync_copy(acc_sh.at[pl.ds(sid * RPS, RPS)], part.at[half, w])
            if deg_pass:
                pltpu.sync_copy(deg_sh.at[pl.ds(sid * RPS, RPS)], degpart.at[w])
            plsc.subcore_barrier()

    return pl.kernel(body, out_type=out_type, mesh=_mesh, scratch_types=scratch,
                     compiler_params=pltpu.CompilerParams(use_tc_tiling_on_sc=False))


_sc_scatter_deg = _sc_scatter(True)
_sc_scatter_nodeg = _sc_scatter(False)


def _mm2_body(x_ref, w0_ref, b0_ref, w1_ref, b1_ref, o0_ref, o1a_ref, o1b_ref):
    x = x_ref[...]
    o0_ref[...] = jnp.dot(x, w0_ref[...], preferred_element_type=jnp.float32) + b0_ref[...]
    o1 = jnp.dot(x, w1_ref[...], preferred_element_type=jnp.float32) + b1_ref[...]
    o1a_ref[...] = o1[:, :DH]
    o1b_ref[...] = o1[:, DH:]


def _mm2(x, w0, b0, w1, b1):
    blk = pl.BlockSpec((BR, D), lambda i: (i, 0))
    hblk = pl.BlockSpec((BR, DH), lambda i: (i, 0))
    wspec = pl.BlockSpec((D, D), lambda i: (0, 0))
    bspec = pl.BlockSpec((1, D), lambda i: (0, 0))
    return pl.pallas_call(
        _mm2_body,
        grid=(NB,),
        in_specs=[blk, wspec, bspec, wspec, bspec],
        out_specs=[blk, hblk, hblk],
        out_shape=[jax.ShapeDtypeStruct((N, D), jnp.float32),
                   jax.ShapeDtypeStruct((N, DH), jnp.float32),
                   jax.ShapeDtypeStruct((N, DH), jnp.float32)],
    )(x, w0, b0.reshape(1, D), w1, b1.reshape(1, D))


def _stats_body(vw0_ref, p_ref, degp_ref, t_ref, sums_ref):
    i = pl.program_id(0)
    deg = degp_ref[0, :, 0] + degp_ref[1, :, 0]
    dinv = 1.0 / (1.0 + deg)
    agg = jnp.concatenate(
        [p_ref[0] + p_ref[1], p_ref[2] + p_ref[3]], axis=1)
    t = (vw0_ref[...] + agg) * dinv[:, None]
    t_ref[...] = t
    s = jnp.sum(t, axis=0)
    s2 = jnp.sum(t * t, axis=0)
    upd = jnp.concatenate(
        [s[None, :], s2[None, :], jnp.zeros((6, D), jnp.float32)], axis=0)

    @pl.when(i == 0)
    def _():
        sums_ref[...] = upd

    @pl.when(i > 0)
    def _():
        sums_ref[...] = sums_ref[...] + upd


def _stats(vw0, part, degpart):
    return pl.pallas_call(
        _stats_body,
        grid=(NB,),
        in_specs=[
            pl.BlockSpec((BR, D), lambda i: (i, 0)),
            pl.BlockSpec((4, BR, DH), lambda i: (0, i, 0)),
            pl.BlockSpec((2, BR, 16), lambda i: (0, i, 0)),
        ],
        out_specs=[
            pl.BlockSpec((BR, D), lambda i: (i, 0)),
            pl.BlockSpec((8, D), lambda i: (0, 0)),
        ],
        out_shape=[
            jax.ShapeDtypeStruct((N, D), jnp.float32),
            jax.ShapeDtypeStruct((8, D), jnp.float32),
        ],
    )(vw0, part, degpart)


def _apply_factory(with_res):
    def body(*refs):
        if with_res:
            t_ref, sums_ref, g_ref, be_ref, res_ref, o_ref = refs
        else:
            t_ref, sums_ref, g_ref, be_ref, o_ref = refs
        m = sums_ref[0, :] / N
        v = sums_ref[1, :] / N - m * m
        scale = g_ref[0, :] * lax.rsqrt(v + EPS)
        y = (t_ref[...] - m[None, :]) * scale[None, :] + be_ref[0, :][None, :]
        if with_res:
            y = y + res_ref[...]
        o_ref[...] = jnp.maximum(y, 0.0)

    blk = pl.BlockSpec((BR, D), lambda i: (i, 0))
    row = pl.BlockSpec((1, D), lambda i: (0, 0))
    srow = pl.BlockSpec((8, D), lambda i: (0, 0))
    in_specs = [blk, srow, row, row] + ([blk] if with_res else [])
    return pl.pallas_call(
        body,
        grid=(NB,),
        in_specs=in_specs,
        out_specs=blk,
        out_shape=jax.ShapeDtypeStruct((N, D), jnp.float32),
    )


_apply_res = _apply_factory(True)
_apply_nores = _apply_factory(False)


def kernel(features, edges, w0_0, b0_0, w1_0, b1_0, g_0, be_0,
           w0_1, b0_1, w1_1, b1_1, g_1, be_1,
           w0_2, b0_2, w1_2, b1_2, g_2, be_2):
    npad = ER * D - 2 * E
    srcs = jnp.concatenate(
        [edges[:, 1], edges[:, 0], jnp.zeros((npad,), jnp.int32)]).reshape(ER, D)
    dsts = jnp.concatenate(
        [edges[:, 0], edges[:, 1], jnp.full((npad,), N, jnp.int32)]).reshape(ER, D)
    zeros = jnp.zeros((RPS, DH), jnp.float32)
    zeros16 = jnp.zeros((RPS, 16), jnp.float32)
    ones16 = jnp.ones((D, 16), jnp.float32)

    x = features
    degpart = None
    params = [(w0_0, b0_0, w1_0, b1_0, g_0, be_0),
              (w0_1, b0_1, w1_1, b1_1, g_1, be_1),
              (w0_2, b0_2, w1_2, b1_2, g_2, be_2)]
    for li, (w0, b0, w1, b1, g, be) in enumerate(params):
        vw0, vw1a, vw1b = _mm2(x, w0, b0, w1, b1)
        if li == 0:
            part, degpart = _sc_scatter_deg(
                vw1a, vw1b, srcs, dsts, zeros, zeros16, ones16)
            degpart = degpart.reshape(2, NPAD, 16)
        else:
            (part,) = _sc_scatter_nodeg(
                vw1a, vw1b, srcs, dsts, zeros, zeros16, ones16)
        t, sums = _stats(vw0, part.reshape(4, NPAD, DH), degpart)
        if li == 2:
            x = _apply_res(t, sums, g.reshape(1, D), be.reshape(1, D), features)
        else:
            x = _apply_nores(t, sums, g.reshape(1, D), be.reshape(1, D))
    return x
